# baseline (speedup 1.0000x reference)
import functools

import jax
import jax.numpy as jnp
from jax import lax
from jax.experimental import pallas as pl
from jax.experimental.pallas import tpu as pltpu
from jax.experimental.pallas import tpu_sc as plsc

F32 = jnp.float32
BF16 = jnp.bfloat16
I32 = jnp.int32

N = 50000
E = 800000
G = 1024
NC, NS = 2, 16
NW = NC * NS
N_PAD = 53248
E_PAD = 819200
HALF = N // 2
AGG_ROWS = HALF + 8
PG_ROWS = 1040

_mesh = functools.partial(
    plsc.VectorSubcoreMesh, core_axis_name="c", subcore_axis_name="s")
_sc_params = pltpu.CompilerParams(use_tc_tiling_on_sc=False)


def _full(spec_shape):
    nd = len(spec_shape)
    return pl.BlockSpec(spec_shape, lambda *_, _n=nd: (0,) * _n)


def _row(spec_shape):
    return pl.BlockSpec(spec_shape, lambda i: (i, 0))


def _tc_prep(h_node_p, T1s_w, T1d_w):
    BN = 4096

    def body(hn_ref, ws_ref, wd_ref, os_ref, od_ref):
        h = hn_ref[...]
        os_ref[...] = jnp.dot(h, ws_ref[...], preferred_element_type=F32)
        od_ref[...] = jnp.dot(h, wd_ref[...], preferred_element_type=F32)

    return pl.pallas_call(
        body,
        grid=(N_PAD // BN,),
        in_specs=[_row((BN, 16)), _full((16, 64)), _full((16, 64))],
        out_specs=[_row((BN, 64))] * 2,
        out_shape=[jax.ShapeDtypeStruct((N_PAD, 64), F32)] * 2,
    )(h_node_p, T1s_w, T1d_w)


def _tc_edge1(h_edge_p, gA, gB, ps, pd, WeP, WeF, wd1, be1):
    BE = 4096

    def body(he_ref, ga_ref, gb_ref, ps_ref, pd_ref, wep_ref, wef_ref,
             wd_ref, b_ref, lo_ref, hi_ref, d_ref):
        h = he_ref[...]
        diff = ps_ref[...] - pd_ref[...]
        d = jnp.sqrt(jnp.sum(diff * diff, axis=1, keepdims=True))
        pre = (jnp.dot(h, wef_ref[...], preferred_element_type=F32)
               + ga_ref[...] + gb_ref[...] + d * wd_ref[...] + b_ref[...])
        he0 = jnp.dot(h, wep_ref[...], preferred_element_type=F32)
        out = he0 + pre * jax.nn.sigmoid(pre)
        lo_ref[...] = out[:, :32]
        hi_ref[...] = out[:, 32:]
        d_ref[...] = jnp.broadcast_to(d, (BE, 8))

    return pl.pallas_call(
        body,
        grid=(E_PAD // BE,),
        in_specs=[_row((BE, 8)), _row((BE, 64)), _row((BE, 64)),
                  _row((BE, 16)), _row((BE, 16)),
                  _full((8, 64)), _full((8, 64)), _full((1, 64)),
                  _full((1, 64))],
        out_specs=[_row((BE, 32)), _row((BE, 32)), _row((BE, 8))],
        out_shape=[jax.ShapeDtypeStruct((E_PAD, 32), F32),
                   jax.ShapeDtypeStruct((E_PAD, 32), F32),
                   jax.ShapeDtypeStruct((E_PAD, 8), F32)],
    )(h_edge_p, gA, gB, ps, pd, WeP, WeF, wd1, be1)


def _tc_edge2(he_lo, he_hi, gA, gB, dcol, We2, wd2, be2):
    BE = 4096

    def body(lo_ref, hi_ref, ga_ref, gb_ref, d_ref, we_ref, wd_ref, b_ref,
             olo_ref, ohi_ref):
        h = jnp.concatenate([lo_ref[...], hi_ref[...]], axis=1)
        d = d_ref[:, :1]
        pre = (jnp.dot(h, we_ref[...], preferred_element_type=F32)
               + ga_ref[...] + gb_ref[...] + d * wd_ref[...] + b_ref[...])
        out = h + pre * jax.nn.sigmoid(pre)
        olo_ref[...] = out[:, :32]
        ohi_ref[...] = out[:, 32:]

    return pl.pallas_call(
        body,
        grid=(E_PAD // BE,),
        in_specs=[_row((BE, 32)), _row((BE, 32)), _row((BE, 64)),
                  _row((BE, 64)), _row((BE, 8)),
                  _full((64, 64)), _full((1, 64)), _full((1, 64))],
        out_specs=[_row((BE, 32)), _row((BE, 32))],
        out_shape=[jax.ShapeDtypeStruct((E_PAD, 32), F32),
                   jax.ShapeDtypeStruct((E_PAD, 32), F32)],
    )(he_lo, he_hi, gA, gB, dcol, We2, wd2, be2)


def _tc_node1(h_node_p, agg_lo, agg_hi, WnWh1, Wa_lo, Wa_hi, bn1,
              Wn, Ws2, Wd2):
    BN = 4096

    def body(hn_ref, al_ref, ah_ref, wh_ref, wal_ref, wah_ref, b_ref,
             wn_ref, ws_ref, wd_ref, o1_ref, os_ref, od_ref):
        h = hn_ref[...]
        pre = (jnp.dot(h, wh_ref[...], preferred_element_type=F32)
               + jnp.dot(al_ref[...], wal_ref[...],
                         preferred_element_type=F32)
               + jnp.dot(ah_ref[...], wah_ref[...],
                         preferred_element_type=F32)
               + b_ref[...])
        hn1 = (jnp.dot(h, wn_ref[...], preferred_element_type=F32)
               + pre * jax.nn.sigmoid(pre))
        o1_ref[...] = hn1
        os_ref[...] = jnp.dot(hn1, ws_ref[...], preferred_element_type=F32)
        od_ref[...] = jnp.dot(hn1, wd_ref[...], preferred_element_type=F32)

    return pl.pallas_call(
        body,
        grid=(N_PAD // BN,),
        in_specs=[_row((BN, 16)), _row((BN, 32)), _row((BN, 32)),
                  _full((16, 64)), _full((32, 64)), _full((32, 64)),
                  _full((1, 64)), _full((16, 64)), _full((64, 64)),
                  _full((64, 64))],
        out_specs=[_row((BN, 64))] * 3,
        out_shape=[jax.ShapeDtypeStruct((N_PAD, 64), F32)] * 3,
    )(h_node_p, agg_lo, agg_hi, WnWh1, Wa_lo, Wa_hi, bn1, Wn, Ws2, Wd2)


def _tc_node2(hn1, agg_lo, agg_hi, Wh2, Wa_lo, Wa_hi, bn2):
    BN = 4096

    def body(hn_ref, al_ref, ah_ref, wh_ref, wal_ref, wah_ref, b_ref,
             out_ref):
        h = hn_ref[...]
        pre = (jnp.dot(h, wh_ref[...], preferred_element_type=F32)
               + jnp.dot(al_ref[...], wal_ref[...],
                         preferred_element_type=F32)
               + jnp.dot(ah_ref[...], wah_ref[...],
                         preferred_element_type=F32)
               + b_ref[...])
        out_ref[...] = h + pre * jax.nn.sigmoid(pre)

    return pl.pallas_call(
        body,
        grid=(N_PAD // BN,),
        in_specs=[_row((BN, 64)), _row((BN, 32)), _row((BN, 32)),
                  _full((64, 64)), _full((32, 64)), _full((32, 64)),
                  _full((1, 64))],
        out_specs=_row((BN, 64)),
        out_shape=jax.ShapeDtypeStruct((N_PAD, 64), F32),
    )(hn1, agg_lo, agg_hi, Wh2, Wa_lo, Wa_hi, bn2)


def _tc_final(pn, cn, pel, peh, ce, Wf1n, W1el, W1eh, bf1, Wf2, bf2):
    def body(pn_ref, cn_ref, pel_ref, peh_ref, ce_ref, w1n_ref, w1el_ref,
             w1eh_ref, b1_ref, w2_ref, b2_ref, out_ref):
        cnt_n = jnp.maximum(cn_ref[:G, :] + cn_ref[G:, :], 1.0)
        cnt_e = jnp.maximum(ce_ref[:G, :] + ce_ref[G:, :], 1.0)
        mean_n = (pn_ref[:G, :] + pn_ref[G:, :]) / cnt_n
        mean_el = (pel_ref[:G, :] + pel_ref[G:, :]) / cnt_e[:, :32]
        mean_eh = (peh_ref[:G, :] + peh_ref[G:, :]) / cnt_e[:, :32]
        h = (jnp.dot(mean_n, w1n_ref[...], preferred_element_type=F32)
             + jnp.dot(mean_el, w1el_ref[...], preferred_element_type=F32)
             + jnp.dot(mean_eh, w1eh_ref[...], preferred_element_type=F32)
             + b1_ref[...])
        h = jnp.maximum(h, 0.0)
        out_ref[...] = (jnp.dot(h, w2_ref[...], preferred_element_type=F32)
                        + b2_ref[...])

    return pl.pallas_call(
        body,
        in_specs=[_full((2 * G, 64)), _full((2 * G, 64)),
                  _full((2 * G, 32)), _full((2 * G, 32)),
                  _full((2 * G, 64)),
                  _full((64, 128)), _full((32, 128)), _full((32, 128)),
                  _full((1, 128)), _full((128, 64)), _full((1, 64))],
        out_specs=_full((G, 64)),
        out_shape=jax.ShapeDtypeStruct((G, 64), F32),
    )(pn, cn, pel, peh, ce, Wf1n, W1el, W1eh, bf1, Wf2, bf2)


_EPW = E_PAD // NW
_NSC = _EPW // 512
_GCH = 256
_NGC = _EPW // _GCH


def _sc_gather(tables, src_p, dst_p):
    n_t = len(tables)
    widths = [w for (_, w, _) in tables]
    sels = [sel for (_, _, sel) in tables]
    data_scr = []
    for w in widths:
        data_scr += [pltpu.VMEM((_GCH, w), F32), pltpu.VMEM((_GCH, w), F32)]

    @functools.partial(
        pl.kernel,
        mesh=_mesh(),
        compiler_params=_sc_params,
        out_type=[jax.ShapeDtypeStruct((E_PAD, w), F32) for w in widths],
        scratch_types=([pltpu.VMEM((_GCH,), I32)] * 4 + data_scr
                       + [pltpu.SemaphoreType.DMA] * 6),
    )
    def k(*refs):
        tabs = refs[:n_t]
        src, dst = refs[n_t], refs[n_t + 1]
        outs = refs[n_t + 2:2 * n_t + 2]
        scr = refs[2 * n_t + 2:]
        IA = [scr[0], scr[1]]
        IB = [scr[2], scr[3]]
        BUF = [[scr[4 + 2 * t], scr[4 + 2 * t + 1]] for t in range(n_t)]
        SI = [scr[4 + 2 * n_t], scr[5 + 2 * n_t]]
        SG = [scr[6 + 2 * n_t], scr[7 + 2 * n_t]]
        SW = [scr[8 + 2 * n_t], scr[9 + 2 * n_t]]
        wid = lax.axis_index("s") * NC + lax.axis_index("c")
        base0 = wid * _EPW

        for b in (0, 1):
            pltpu.async_copy(src.at[pl.ds(base0 + b * _GCH, _GCH)],
                             IA[b], SI[b])
            pltpu.async_copy(dst.at[pl.ds(base0 + b * _GCH, _GCH)],
                             IB[b], SI[b])

        def pair(p, carry):
            for b in (0, 1):
                j = p * 2 + b

                @pl.when(p > 0)
                def _():
                    for t in range(n_t):
                        pltpu.make_async_copy(
                            BUF[t][b], outs[t].at[pl.ds(base0, _GCH)],
                            SW[b]).wait()

                pltpu.make_async_copy(src.at[pl.ds(base0, _GCH)],
                                      IA[b], SI[b]).wait()
                pltpu.make_async_copy(dst.at[pl.ds(base0, _GCH)],
                                      IB[b], SI[b]).wait()
                cps = []
                for t in range(n_t):
                    idx = IA[b] if sels[t] == 0 else IB[b]
                    for kk in range(_GCH // 128):
                        sl = pl.ds(kk * 128, 128)
                        cps.append(pltpu.async_copy(
                            tabs[t].at[idx.at[sl]], BUF[t][b].at[sl], SG[b]))
                for cp in cps:
                    cp.wait()
                nxt = base0 + jnp.minimum(j + 2, _NGC - 1) * _GCH
                pltpu.async_copy(src.at[pl.ds(nxt, _GCH)], IA[b], SI[b])
                pltpu.async_copy(dst.at[pl.ds(nxt, _GCH)], IB[b], SI[b])
                base = base0 + j * _GCH
                for t in range(n_t):
                    pltpu.async_copy(BUF[t][b], outs[t].at[pl.ds(base, _GCH)],
                                     SW[b])
            return carry

        lax.fori_loop(0, _NGC // 2, pair, 0)
        for b in (0, 1):
            for t in range(n_t):
                pltpu.make_async_copy(BUF[t][b],
                                      outs[t].at[pl.ds(base0, _GCH)],
                                      SW[b]).wait()
            pltpu.make_async_copy(src.at[pl.ds(base0, _GCH)],
                                  IA[b], SI[b]).wait()
            pltpu.make_async_copy(dst.at[pl.ds(base0, _GCH)],
                                  IB[b], SI[b]).wait()

    return k(*[t for (t, _, _) in tables], src_p, dst_p)


def _sc_scatter(he_half, dloc_flat, zrows32):
    @functools.partial(
        pl.kernel,
        mesh=_mesh(),
        compiler_params=_sc_params,
        out_type=jax.ShapeDtypeStruct((N_PAD, 32), F32),
        scratch_types=[pltpu.VMEM((4, 128), I32), pltpu.VMEM((4, 128), I32),
                       pltpu.VMEM((512, 32), F32), pltpu.VMEM((512, 32), F32),
                       pltpu.VMEM_SHARED((AGG_ROWS, 32), F32),
                       pltpu.SemaphoreType.DMA, pltpu.SemaphoreType.DMA,
                       pltpu.SemaphoreType.DMA, pltpu.SemaphoreType.DMA],
    )
    def k(he_ref, dloc, zeros, agg, ix0, ix1, hv0, hv1, shared,
          sl0, sl1, sa0, sa1):
        c = lax.axis_index("c")
        s = lax.axis_index("s")
        IX, HV = [ix0, ix1], [hv0, hv1]
        SL, SA = [sl0, sl1], [sa0, sa1]
        pltpu.sync_copy(zeros.at[pl.ds(0, 1563)],
                        shared.at[pl.ds(s * 1563, 1563)])
        plsc.subcore_barrier()
        gbase0 = c * (E_PAD // 128) + s * 400
        ebase0 = s * (400 * 128)

        for b in (0, 1):
            pltpu.async_copy(he_ref.at[pl.ds(ebase0 + b * 512, 512)],
                             HV[b], SL[b])
            pltpu.async_copy(dloc.at[pl.ds(gbase0 + b * 4, 4)], IX[b], SL[b])

        def pair(p, carry):
            for b in (0, 1):
                j = p * 2 + b
                pltpu.make_async_copy(he_ref.at[pl.ds(ebase0, 512)],
                                      HV[b], SL[b]).wait()
                pltpu.make_async_copy(dloc.at[pl.ds(gbase0, 4)],
                                      IX[b], SL[b]).wait()
                cps = [pltpu.async_copy(HV[b].at[pl.ds(kk * 128, 128)],
                                        shared.at[IX[b].at[kk]], SA[b],
                                        add=True)
                       for kk in range(4)]
                for cp in cps:
                    cp.wait()
                nj = jnp.minimum(j + 2, 99)
                pltpu.async_copy(he_ref.at[pl.ds(ebase0 + nj * 512, 512)],
                                 HV[b], SL[b])
                pltpu.async_copy(dloc.at[pl.ds(gbase0 + nj * 4, 4)],
                                 IX[b], SL[b])
            return carry

        lax.fori_loop(0, 50, pair, 0)
        for b in (0, 1):
            pltpu.make_async_copy(he_ref.at[pl.ds(ebase0, 512)],
                                  HV[b], SL[b]).wait()
            pltpu.make_async_copy(dloc.at[pl.ds(gbase0, 4)],
                                  IX[b], SL[b]).wait()
        plsc.subcore_barrier()
        rbase = c * HALF + s * 1563

        @pl.when(s < NS - 1)
        def _():
            pltpu.sync_copy(shared.at[pl.ds(s * 1563, 1563)],
                            agg.at[pl.ds(rbase, 1563)])

        @pl.when(s == NS - 1)
        def _():
            pltpu.sync_copy(shared.at[pl.ds(s * 1563, 1555)],
                            agg.at[pl.ds(rbase, 1555)])

    return k(he_half, dloc_flat, zrows32)


def _sc_pool(hn2, he_lo, he_hi, bn2d, be2d, zrows, zrows32, ones):
    NPW = N_PAD // NW
    GS = G // NS

    @functools.partial(
        pl.kernel,
        mesh=_mesh(),
        compiler_params=_sc_params,
        out_type=[jax.ShapeDtypeStruct((2 * G, 64), F32),
                  jax.ShapeDtypeStruct((2 * G, 64), F32),
                  jax.ShapeDtypeStruct((2 * G, 32), F32),
                  jax.ShapeDtypeStruct((2 * G, 32), F32),
                  jax.ShapeDtypeStruct((2 * G, 64), F32)],
        scratch_types=[pltpu.VMEM((4, 128), I32), pltpu.VMEM((4, 128), I32),
                       pltpu.VMEM((128, 64), F32),
                       pltpu.VMEM((512, 32), F32), pltpu.VMEM((512, 32), F32),
                       pltpu.VMEM((512, 32), F32), pltpu.VMEM((512, 32), F32),
                       pltpu.VMEM((128, 64), F32),
                       pltpu.VMEM_SHARED((PG_ROWS, 64), F32),
                       pltpu.VMEM_SHARED((PG_ROWS, 64), F32),
                       pltpu.VMEM_SHARED((PG_ROWS, 32), F32),
                       pltpu.VMEM_SHARED((PG_ROWS, 32), F32),
                       pltpu.VMEM_SHARED((PG_ROWS, 64), F32),
                       pltpu.SemaphoreType.DMA, pltpu.SemaphoreType.DMA,
                       pltpu.SemaphoreType.DMA, pltpu.SemaphoreType.DMA],
    )
    def k(hn_ref, helo_ref, hehi_ref, bn_ref, be_ref, zeros, zeros32,
          ones_ref, o_sn, o_cn, o_sel, o_seh, o_ce,
          ix0, ix1, datn, dl0, dl1, dh0, dh1, onesv,
          sn, cn, sel, seh, ce, sl0, sl1, sa0, sa1):
        c = lax.axis_index("c")
        s = lax.axis_index("s")
        wid = s * NC + c
        IX, DL, DH = [ix0, ix1], [dl0, dl1], [dh0, dh1]
        SL, SA = [sl0, sl1], [sa0, sa1]
        for buf in (sn, cn, ce):
            pltpu.sync_copy(zeros.at[pl.ds(0, 65)],
                            buf.at[pl.ds(s * 65, 65)])
        for buf in (sel, seh):
            pltpu.sync_copy(zeros32.at[pl.ds(0, 65)],
                            buf.at[pl.ds(s * 65, 65)])
        pltpu.sync_copy(ones_ref.at[pl.ds(0, 128)], onesv)
        plsc.subcore_barrier()

        ebase0 = wid * _EPW
        egrp0 = wid * (_EPW // 128)

        for b in (0, 1):
            pltpu.async_copy(helo_ref.at[pl.ds(ebase0 + b * 512, 512)],
                             DL[b], SL[b])
            pltpu.async_copy(hehi_ref.at[pl.ds(ebase0 + b * 512, 512)],
                             DH[b], SL[b])
            pltpu.async_copy(be_ref.at[pl.ds(egrp0 + b * 4, 4)], IX[b], SL[b])

        def epair(p, carry):
            for b in (0, 1):
                j = p * 2 + b
                pltpu.make_async_copy(helo_ref.at[pl.ds(ebase0, 512)],
                                      DL[b], SL[b]).wait()
                pltpu.make_async_copy(hehi_ref.at[pl.ds(ebase0, 512)],
                                      DH[b], SL[b]).wait()
                pltpu.make_async_copy(be_ref.at[pl.ds(egrp0, 4)],
                                      IX[b], SL[b]).wait()
                cps = []
                for kk in range(4):
                    cps.append(pltpu.async_copy(
                        DL[b].at[pl.ds(kk * 128, 128)],
                        sel.at[IX[b].at[kk]], SA[b], add=True))
                    cps.append(pltpu.async_copy(
                        DH[b].at[pl.ds(kk * 128, 128)],
                        seh.at[IX[b].at[kk]], SA[b], add=True))
                    cps.append(pltpu.async_copy(
                        onesv, ce.at[IX[b].at[kk]], SA[b], add=True))
                for cp in cps:
                    cp.wait()
                nj = jnp.minimum(j + 2, _NSC - 1)
                pltpu.async_copy(helo_ref.at[pl.ds(ebase0 + nj * 512, 512)],
                                 DL[b], SL[b])
                pltpu.async_copy(hehi_ref.at[pl.ds(ebase0 + nj * 512, 512)],
                                 DH[b], SL[b])
                pltpu.async_copy(be_ref.at[pl.ds(egrp0 + nj * 4, 4)],
                                 IX[b], SL[b])
            return carry

        lax.fori_loop(0, _NSC // 2, epair, 0)
        for b in (0, 1):
            pltpu.make_async_copy(helo_ref.at[pl.ds(ebase0, 512)],
                                  DL[b], SL[b]).wait()
            pltpu.make_async_copy(hehi_ref.at[pl.ds(ebase0, 512)],
                                  DH[b], SL[b]).wait()
            pltpu.make_async_copy(be_ref.at[pl.ds(egrp0, 4)],
                                  IX[b], SL[b]).wait()

        nbase0 = wid * NPW
        ngrp0 = wid * (NPW // 128)

        def nstep(j, carry):
            pltpu.sync_copy(hn_ref.at[pl.ds(nbase0 + j * 128, 128)], datn)
            pltpu.sync_copy(bn_ref.at[pl.ds(ngrp0 + j, 1)],
                            ix0.at[pl.ds(0, 1)])
            cp1 = pltpu.async_copy(datn, sn.at[ix0.at[0]], sa0, add=True)
            cp2 = pltpu.async_copy(onesv, cn.at[ix0.at[0]], sa0, add=True)
            cp1.wait()
            cp2.wait()
            return carry

        lax.fori_loop(0, NPW // 128, nstep, 0)
        plsc.subcore_barrier()
        rbase = c * G + s * GS
        pltpu.sync_copy(sn.at[pl.ds(s * GS, GS)], o_sn.at[pl.ds(rbase, GS)])
        pltpu.sync_copy(cn.at[pl.ds(s * GS, GS)], o_cn.at[pl.ds(rbase, GS)])
        pltpu.sync_copy(sel.at[pl.ds(s * GS, GS)], o_sel.at[pl.ds(rbase, GS)])
        pltpu.sync_copy(seh.at[pl.ds(s * GS, GS)], o_seh.at[pl.ds(rbase, GS)])
        pltpu.sync_copy(ce.at[pl.ds(s * GS, GS)], o_ce.at[pl.ds(rbase, GS)])

    return k(hn2, he_lo, he_hi, bn2d, be2d, zrows, zrows32, ones)


def kernel(h_node, pos_node, batch_node, h_edge, edge_index, batch_edge,
           Wn, We, Wem, bem, Wnu, bnu, Wf1, bf1, Wf2, bf2):
    src = edge_index[0].astype(I32)
    dst = edge_index[1].astype(I32)
    src_p = jnp.concatenate([src, jnp.zeros((E_PAD - E,), I32)])
    dst_p = jnp.concatenate([dst, jnp.full((E_PAD - E,), N, I32)])
    dloc0 = jnp.where(dst_p < HALF, dst_p, HALF)
    dloc1 = jnp.where((dst_p >= HALF) & (dst_p < N), dst_p - HALF, HALF)
    dloc_flat = jnp.concatenate([dloc0, dloc1]).reshape(2 * (E_PAD // 128),
                                                        128)
    be2d = jnp.concatenate(
        [batch_edge.astype(I32), jnp.full((E_PAD - E,), G, I32)]
    ).reshape(E_PAD // 128, 128)
    bn2d = jnp.concatenate(
        [batch_node.astype(I32), jnp.full((N_PAD - N,), G, I32)]
    ).reshape(N_PAD // 128, 128)

    h_edge_p = jnp.pad(h_edge, ((0, E_PAD - E), (0, 3)))
    h_node_p = jnp.pad(h_node, ((0, N_PAD - N), (0, 0)))
    pos_p = jnp.pad(pos_node, ((0, 0), (0, 13)))
    zrows = jnp.zeros((1568, 64), F32)
    zrows32 = jnp.zeros((1568, 32), F32)
    ones = jnp.ones((128, 64), F32)

    We1, Ws1, Wd1 = Wem[0, :64], Wem[0, 64:128], Wem[0, 128:192]
    wd1 = Wem[0, 192:193]
    We2, Ws2, Wd2 = Wem[1, :64], Wem[1, 64:128], Wem[1, 128:192]
    wd2 = Wem[1, 192:193]
    Wh1, Wa1 = Wnu[0, :64], Wnu[0, 64:128]
    Wh2, Wa2 = Wnu[1, :64], Wnu[1, 64:128]
    WeP = jnp.pad(We, ((0, 3), (0, 0)))
    WeF1 = jnp.pad(We @ We1, ((0, 3), (0, 0)))
    be1 = bem[0:1]
    be2 = bem[1:2]
    bn1 = bnu[0:1]
    bn2 = bnu[1:2]
    bf1r = bf1[None, :]
    bf2r = bf2[None, :]

    T1s, T1d = _tc_prep(h_node_p, Wn @ Ws1, Wn @ Wd1)
    gA1, gB1, ps, pd = _sc_gather(
        [(T1s, 64, 0), (T1d, 64, 1), (pos_p, 16, 0), (pos_p, 16, 1)],
        src_p, dst_p)
    he1_lo, he1_hi, dcol = _tc_edge1(h_edge_p, gA1, gB1, ps, pd,
                                     WeP, WeF1, wd1, be1)
    agg1_lo = _sc_scatter(he1_lo, dloc_flat, zrows32)
    agg1_hi = _sc_scatter(he1_hi, dloc_flat, zrows32)
    hn1, T2s, T2d = _tc_node1(h_node_p, agg1_lo, agg1_hi, Wn @ Wh1,
                              Wa1[:32], Wa1[32:], bn1, Wn, Ws2, Wd2)
    gA2, gB2 = _sc_gather([(T2s, 64, 0), (T2d, 64, 1)], src_p, dst_p)
    he2_lo, he2_hi = _tc_edge2(he1_lo, he1_hi, gA2, gB2, dcol,
                               We2, wd2, be2)
    agg2_lo = _sc_scatter(he2_lo, dloc_flat, zrows32)
    agg2_hi = _sc_scatter(he2_hi, dloc_flat, zrows32)
    hn2 = _tc_node2(hn1, agg2_lo, agg2_hi, Wh2, Wa2[:32], Wa2[32:], bn2)

    pn, cn, pel, peh, ce = _sc_pool(hn2, he2_lo, he2_hi, bn2d, be2d,
                                    zrows, zrows32, ones)
    emb = _tc_final(pn, cn, pel, peh, ce, Wf1[:64],
                    Wf1[64:96], Wf1[96:], bf1r, Wf2, bf2r)
    return (emb, batch_node)


# speedup vs baseline: 1.0665x; 1.0665x over previous
import functools

import jax
import jax.numpy as jnp
from jax import lax
from jax.experimental import pallas as pl
from jax.experimental.pallas import tpu as pltpu
from jax.experimental.pallas import tpu_sc as plsc

F32 = jnp.float32
BF16 = jnp.bfloat16
I32 = jnp.int32

N = 50000
E = 800000
G = 1024
NC, NS = 2, 16
NW = NC * NS
N_PAD = 53248
E_PAD = 819200
HALF = N // 2
AGG_ROWS = HALF + 8
PG_ROWS = 1040

_mesh = functools.partial(
    plsc.VectorSubcoreMesh, core_axis_name="c", subcore_axis_name="s")
_sc_params = pltpu.CompilerParams(use_tc_tiling_on_sc=False)


def _full(spec_shape):
    nd = len(spec_shape)
    return pl.BlockSpec(spec_shape, lambda *_, _n=nd: (0,) * _n)


def _row(spec_shape):
    return pl.BlockSpec(spec_shape, lambda i: (i, 0))


def _tc_prep(h_node_p, pos_p, T1s_w, T1d_w):
    BN = 4096

    def body(hn_ref, pos_ref, ws_ref, wd_ref, os_ref, od_ref):
        h = hn_ref[...]
        p = pos_ref[...]
        ts = jnp.dot(h, ws_ref[...], preferred_element_type=F32)
        td = jnp.dot(h, wd_ref[...], preferred_element_type=F32)
        os_ref[...] = jnp.concatenate([ts, p], axis=1).astype(BF16)
        od_ref[...] = jnp.concatenate([td, p], axis=1).astype(BF16)

    return pl.pallas_call(
        body,
        grid=(N_PAD // BN,),
        in_specs=[_row((BN, 16)), _row((BN, 16)),
                  _full((16, 64)), _full((16, 64))],
        out_specs=[_row((BN, 80))] * 2,
        out_shape=[jax.ShapeDtypeStruct((N_PAD, 80), BF16)] * 2,
    )(h_node_p, pos_p, T1s_w, T1d_w)


def _tc_edge1(h_edge_p, gA, gB, WeP, WeF, wd1, be1):
    BE = 4096

    def body(he_ref, ga_ref, gb_ref, wep_ref, wef_ref,
             wd_ref, b_ref, lo_ref, hi_ref, d_ref):
        h = he_ref[...]
        ga = ga_ref[...].astype(F32)
        gb = gb_ref[...].astype(F32)
        diff = ga[:, 64:] - gb[:, 64:]
        d = jnp.sqrt(jnp.sum(diff * diff, axis=1, keepdims=True))
        pre = (jnp.dot(h, wef_ref[...], preferred_element_type=F32)
               + ga[:, :64] + gb[:, :64] + d * wd_ref[...] + b_ref[...])
        he0 = jnp.dot(h, wep_ref[...], preferred_element_type=F32)
        out = he0 + pre * jax.nn.sigmoid(pre)
        lo_ref[...] = out[:, :32]
        hi_ref[...] = out[:, 32:]
        d_ref[...] = jnp.broadcast_to(d, (BE, 8))

    return pl.pallas_call(
        body,
        grid=(E_PAD // BE,),
        in_specs=[_row((BE, 8)), _row((BE, 80)), _row((BE, 80)),
                  _full((8, 64)), _full((8, 64)), _full((1, 64)),
                  _full((1, 64))],
        out_specs=[_row((BE, 32)), _row((BE, 32)), _row((BE, 8))],
        out_shape=[jax.ShapeDtypeStruct((E_PAD, 32), F32),
                   jax.ShapeDtypeStruct((E_PAD, 32), F32),
                   jax.ShapeDtypeStruct((E_PAD, 8), F32)],
    )(h_edge_p, gA, gB, WeP, WeF, wd1, be1)


def _tc_edge2(he_lo, he_hi, gA, gB, dcol, We2, wd2, be2):
    BE = 4096

    def body(lo_ref, hi_ref, ga_ref, gb_ref, d_ref, we_ref, wd_ref, b_ref,
             olo_ref, ohi_ref):
        h = jnp.concatenate([lo_ref[...], hi_ref[...]], axis=1)
        d = d_ref[:, :1]
        pre = (jnp.dot(h, we_ref[...], preferred_element_type=F32)
               + ga_ref[...].astype(F32) + gb_ref[...].astype(F32)
               + d * wd_ref[...] + b_ref[...])
        out = h + pre * jax.nn.sigmoid(pre)
        olo_ref[...] = out[:, :32]
        ohi_ref[...] = out[:, 32:]

    return pl.pallas_call(
        body,
        grid=(E_PAD // BE,),
        in_specs=[_row((BE, 32)), _row((BE, 32)), _row((BE, 64)),
                  _row((BE, 64)), _row((BE, 8)),
                  _full((64, 64)), _full((1, 64)), _full((1, 64))],
        out_specs=[_row((BE, 32)), _row((BE, 32))],
        out_shape=[jax.ShapeDtypeStruct((E_PAD, 32), F32),
                   jax.ShapeDtypeStruct((E_PAD, 32), F32)],
    )(he_lo, he_hi, gA, gB, dcol, We2, wd2, be2)


def _tc_node1(h_node_p, agg_lo, agg_hi, WnWh1, Wa_lo, Wa_hi, bn1,
              Wn, Ws2, Wd2):
    BN = 4096

    def body(hn_ref, al_ref, ah_ref, wh_ref, wal_ref, wah_ref, b_ref,
             wn_ref, ws_ref, wd_ref, o1_ref, os_ref, od_ref):
        h = hn_ref[...]
        pre = (jnp.dot(h, wh_ref[...], preferred_element_type=F32)
               + jnp.dot(al_ref[...], wal_ref[...],
                         preferred_element_type=F32)
               + jnp.dot(ah_ref[...], wah_ref[...],
                         preferred_element_type=F32)
               + b_ref[...])
        hn1 = (jnp.dot(h, wn_ref[...], preferred_element_type=F32)
               + pre * jax.nn.sigmoid(pre))
        o1_ref[...] = hn1
        os_ref[...] = jnp.dot(hn1, ws_ref[...],
                              preferred_element_type=F32).astype(BF16)
        od_ref[...] = jnp.dot(hn1, wd_ref[...],
                              preferred_element_type=F32).astype(BF16)

    return pl.pallas_call(
        body,
        grid=(N_PAD // BN,),
        in_specs=[_row((BN, 16)), _row((BN, 32)), _row((BN, 32)),
                  _full((16, 64)), _full((32, 64)), _full((32, 64)),
                  _full((1, 64)), _full((16, 64)), _full((64, 64)),
                  _full((64, 64))],
        out_specs=[_row((BN, 64))] * 3,
        out_shape=[jax.ShapeDtypeStruct((N_PAD, 64), F32),
                   jax.ShapeDtypeStruct((N_PAD, 64), BF16),
                   jax.ShapeDtypeStruct((N_PAD, 64), BF16)],
    )(h_node_p, agg_lo, agg_hi, WnWh1, Wa_lo, Wa_hi, bn1, Wn, Ws2, Wd2)


def _tc_node2(hn1, agg_lo, agg_hi, Wh2, Wa_lo, Wa_hi, bn2):
    BN = 4096

    def body(hn_ref, al_ref, ah_ref, wh_ref, wal_ref, wah_ref, b_ref,
             out_ref):
        h = hn_ref[...]
        pre = (jnp.dot(h, wh_ref[...], preferred_element_type=F32)
               + jnp.dot(al_ref[...], wal_ref[...],
                         preferred_element_type=F32)
               + jnp.dot(ah_ref[...], wah_ref[...],
                         preferred_element_type=F32)
               + b_ref[...])
        out_ref[...] = h + pre * jax.nn.sigmoid(pre)

    return pl.pallas_call(
        body,
        grid=(N_PAD // BN,),
        in_specs=[_row((BN, 64)), _row((BN, 32)), _row((BN, 32)),
                  _full((64, 64)), _full((32, 64)), _full((32, 64)),
                  _full((1, 64))],
        out_specs=_row((BN, 64)),
        out_shape=jax.ShapeDtypeStruct((N_PAD, 64), F32),
    )(hn1, agg_lo, agg_hi, Wh2, Wa_lo, Wa_hi, bn2)


def _tc_final(pn, cn, pel, peh, ce, Wf1n, W1el, W1eh, bf1, Wf2, bf2):
    def body(pn_ref, cn_ref, pel_ref, peh_ref, ce_ref, w1n_ref, w1el_ref,
             w1eh_ref, b1_ref, w2_ref, b2_ref, out_ref):
        cnt_n = jnp.maximum(cn_ref[:G, :] + cn_ref[G:, :], 1.0)
        cnt_e = jnp.maximum(ce_ref[:G, :] + ce_ref[G:, :], 1.0)
        mean_n = (pn_ref[:G, :] + pn_ref[G:, :]) / cnt_n
        mean_el = (pel_ref[:G, :] + pel_ref[G:, :]) / cnt_e[:, :32]
        mean_eh = (peh_ref[:G, :] + peh_ref[G:, :]) / cnt_e[:, :32]
        h = (jnp.dot(mean_n, w1n_ref[...], preferred_element_type=F32)
             + jnp.dot(mean_el, w1el_ref[...], preferred_element_type=F32)
             + jnp.dot(mean_eh, w1eh_ref[...], preferred_element_type=F32)
             + b1_ref[...])
        h = jnp.maximum(h, 0.0)
        out_ref[...] = (jnp.dot(h, w2_ref[...], preferred_element_type=F32)
                        + b2_ref[...])

    return pl.pallas_call(
        body,
        in_specs=[_full((2 * G, 64)), _full((2 * G, 64)),
                  _full((2 * G, 32)), _full((2 * G, 32)),
                  _full((2 * G, 64)),
                  _full((64, 128)), _full((32, 128)), _full((32, 128)),
                  _full((1, 128)), _full((128, 64)), _full((1, 64))],
        out_specs=_full((G, 64)),
        out_shape=jax.ShapeDtypeStruct((G, 64), F32),
    )(pn, cn, pel, peh, ce, Wf1n, W1el, W1eh, bf1, Wf2, bf2)


_EPW = E_PAD // NW
_NSC = _EPW // 512
_GCH = 256
_NGC = _EPW // _GCH


def _sc_gather(tables, src_p, dst_p):
    n_t = len(tables)
    widths = [w for (_, w, _, _) in tables]
    dts = [dt for (_, _, dt, _) in tables]
    sels = [sel for (_, _, _, sel) in tables]
    data_scr = []
    for w, dt in zip(widths, dts):
        data_scr += [pltpu.VMEM((_GCH, w), dt), pltpu.VMEM((_GCH, w), dt)]

    @functools.partial(
        pl.kernel,
        mesh=_mesh(),
        compiler_params=_sc_params,
        out_type=[jax.ShapeDtypeStruct((E_PAD, w), dt)
                  for w, dt in zip(widths, dts)],
        scratch_types=([pltpu.VMEM((_GCH,), I32)] * 4 + data_scr
                       + [pltpu.SemaphoreType.DMA] * 6),
    )
    def k(*refs):
        tabs = refs[:n_t]
        src, dst = refs[n_t], refs[n_t + 1]
        outs = refs[n_t + 2:2 * n_t + 2]
        scr = refs[2 * n_t + 2:]
        IA = [scr[0], scr[1]]
        IB = [scr[2], scr[3]]
        BUF = [[scr[4 + 2 * t], scr[4 + 2 * t + 1]] for t in range(n_t)]
        SI = [scr[4 + 2 * n_t], scr[5 + 2 * n_t]]
        SG = [scr[6 + 2 * n_t], scr[7 + 2 * n_t]]
        SW = [scr[8 + 2 * n_t], scr[9 + 2 * n_t]]
        wid = lax.axis_index("s") * NC + lax.axis_index("c")
        base0 = wid * _EPW

        for b in (0, 1):
            pltpu.async_copy(src.at[pl.ds(base0 + b * _GCH, _GCH)],
                             IA[b], SI[b])
            pltpu.async_copy(dst.at[pl.ds(base0 + b * _GCH, _GCH)],
                             IB[b], SI[b])

        def pair(p, carry):
            for b in (0, 1):
                j = p * 2 + b

                @pl.when(p > 0)
                def _():
                    for t in range(n_t):
                        pltpu.make_async_copy(
                            BUF[t][b], outs[t].at[pl.ds(base0, _GCH)],
                            SW[b]).wait()

                pltpu.make_async_copy(src.at[pl.ds(base0, _GCH)],
                                      IA[b], SI[b]).wait()
                pltpu.make_async_copy(dst.at[pl.ds(base0, _GCH)],
                                      IB[b], SI[b]).wait()
                cps = []
                for t in range(n_t):
                    idx = IA[b] if sels[t] == 0 else IB[b]
                    for kk in range(_GCH // 128):
                        sl = pl.ds(kk * 128, 128)
                        cps.append(pltpu.async_copy(
                            tabs[t].at[idx.at[sl]], BUF[t][b].at[sl], SG[b]))
                for cp in cps:
                    cp.wait()
                nxt = base0 + jnp.minimum(j + 2, _NGC - 1) * _GCH
                pltpu.async_copy(src.at[pl.ds(nxt, _GCH)], IA[b], SI[b])
                pltpu.async_copy(dst.at[pl.ds(nxt, _GCH)], IB[b], SI[b])
                base = base0 + j * _GCH
                for t in range(n_t):
                    pltpu.async_copy(BUF[t][b], outs[t].at[pl.ds(base, _GCH)],
                                     SW[b])
            return carry

        lax.fori_loop(0, _NGC // 2, pair, 0)
        for b in (0, 1):
            for t in range(n_t):
                pltpu.make_async_copy(BUF[t][b],
                                      outs[t].at[pl.ds(base0, _GCH)],
                                      SW[b]).wait()
            pltpu.make_async_copy(src.at[pl.ds(base0, _GCH)],
                                  IA[b], SI[b]).wait()
            pltpu.make_async_copy(dst.at[pl.ds(base0, _GCH)],
                                  IB[b], SI[b]).wait()

    return k(*[t for (t, _, _, _) in tables], src_p, dst_p)


def _sc_scatter(he_half, dloc_flat, zrows32):
    @functools.partial(
        pl.kernel,
        mesh=_mesh(),
        compiler_params=_sc_params,
        out_type=jax.ShapeDtypeStruct((N_PAD, 32), F32),
        scratch_types=[pltpu.VMEM((4, 128), I32), pltpu.VMEM((4, 128), I32),
                       pltpu.VMEM((512, 32), F32), pltpu.VMEM((512, 32), F32),
                       pltpu.VMEM_SHARED((AGG_ROWS, 32), F32),
                       pltpu.SemaphoreType.DMA, pltpu.SemaphoreType.DMA,
                       pltpu.SemaphoreType.DMA, pltpu.SemaphoreType.DMA],
    )
    def k(he_ref, dloc, zeros, agg, ix0, ix1, hv0, hv1, shared,
          sl0, sl1, sa0, sa1):
        c = lax.axis_index("c")
        s = lax.axis_index("s")
        IX, HV = [ix0, ix1], [hv0, hv1]
        SL, SA = [sl0, sl1], [sa0, sa1]
        pltpu.sync_copy(zeros.at[pl.ds(0, 1563)],
                        shared.at[pl.ds(s * 1563, 1563)])
        plsc.subcore_barrier()
        gbase0 = c * (E_PAD // 128) + s * 400
        ebase0 = s * (400 * 128)

        for b in (0, 1):
            pltpu.async_copy(he_ref.at[pl.ds(ebase0 + b * 512, 512)],
                             HV[b], SL[b])
            pltpu.async_copy(dloc.at[pl.ds(gbase0 + b * 4, 4)], IX[b], SL[b])

        def pair(p, carry):
            for b in (0, 1):
                j = p * 2 + b
                pltpu.make_async_copy(he_ref.at[pl.ds(ebase0, 512)],
                                      HV[b], SL[b]).wait()
                pltpu.make_async_copy(dloc.at[pl.ds(gbase0, 4)],
                                      IX[b], SL[b]).wait()
                cps = [pltpu.async_copy(HV[b].at[pl.ds(kk * 128, 128)],
                                        shared.at[IX[b].at[kk]], SA[b],
                                        add=True)
                       for kk in range(4)]
                for cp in cps:
                    cp.wait()
                nj = jnp.minimum(j + 2, 99)
                pltpu.async_copy(he_ref.at[pl.ds(ebase0 + nj * 512, 512)],
                                 HV[b], SL[b])
                pltpu.async_copy(dloc.at[pl.ds(gbase0 + nj * 4, 4)],
                                 IX[b], SL[b])
            return carry

        lax.fori_loop(0, 50, pair, 0)
        for b in (0, 1):
            pltpu.make_async_copy(he_ref.at[pl.ds(ebase0, 512)],
                                  HV[b], SL[b]).wait()
            pltpu.make_async_copy(dloc.at[pl.ds(gbase0, 4)],
                                  IX[b], SL[b]).wait()
        plsc.subcore_barrier()
        rbase = c * HALF + s * 1563

        @pl.when(s < NS - 1)
        def _():
            pltpu.sync_copy(shared.at[pl.ds(s * 1563, 1563)],
                            agg.at[pl.ds(rbase, 1563)])

        @pl.when(s == NS - 1)
        def _():
            pltpu.sync_copy(shared.at[pl.ds(s * 1563, 1555)],
                            agg.at[pl.ds(rbase, 1555)])

    return k(he_half, dloc_flat, zrows32)


def _sc_pool(hn2, he_lo, he_hi, bn2d, be2d, zrows, zrows32, ones):
    NPW = N_PAD // NW
    GS = G // NS

    @functools.partial(
        pl.kernel,
        mesh=_mesh(),
        compiler_params=_sc_params,
        out_type=[jax.ShapeDtypeStruct((2 * G, 64), F32),
                  jax.ShapeDtypeStruct((2 * G, 64), F32),
                  jax.ShapeDtypeStruct((2 * G, 32), F32),
                  jax.ShapeDtypeStruct((2 * G, 32), F32),
                  jax.ShapeDtypeStruct((2 * G, 64), F32)],
        scratch_types=[pltpu.VMEM((4, 128), I32), pltpu.VMEM((4, 128), I32),
                       pltpu.VMEM((128, 64), F32),
                       pltpu.VMEM((512, 32), F32), pltpu.VMEM((512, 32), F32),
                       pltpu.VMEM((512, 32), F32), pltpu.VMEM((512, 32), F32),
                       pltpu.VMEM((128, 64), F32),
                       pltpu.VMEM_SHARED((PG_ROWS, 64), F32),
                       pltpu.VMEM_SHARED((PG_ROWS, 64), F32),
                       pltpu.VMEM_SHARED((PG_ROWS, 32), F32),
                       pltpu.VMEM_SHARED((PG_ROWS, 32), F32),
                       pltpu.VMEM_SHARED((PG_ROWS, 64), F32),
                       pltpu.SemaphoreType.DMA, pltpu.SemaphoreType.DMA,
                       pltpu.SemaphoreType.DMA, pltpu.SemaphoreType.DMA],
    )
    def k(hn_ref, helo_ref, hehi_ref, bn_ref, be_ref, zeros, zeros32,
          ones_ref, o_sn, o_cn, o_sel, o_seh, o_ce,
          ix0, ix1, datn, dl0, dl1, dh0, dh1, onesv,
          sn, cn, sel, seh, ce, sl0, sl1, sa0, sa1):
        c = lax.axis_index("c")
        s = lax.axis_index("s")
        wid = s * NC + c
        IX, DL, DH = [ix0, ix1], [dl0, dl1], [dh0, dh1]
        SL, SA = [sl0, sl1], [sa0, sa1]
        for buf in (sn, cn, ce):
            pltpu.sync_copy(zeros.at[pl.ds(0, 65)],
                            buf.at[pl.ds(s * 65, 65)])
        for buf in (sel, seh):
            pltpu.sync_copy(zeros32.at[pl.ds(0, 65)],
                            buf.at[pl.ds(s * 65, 65)])
        pltpu.sync_copy(ones_ref.at[pl.ds(0, 128)], onesv)
        plsc.subcore_barrier()

        ebase0 = wid * _EPW
        egrp0 = wid * (_EPW // 128)

        for b in (0, 1):
            pltpu.async_copy(helo_ref.at[pl.ds(ebase0 + b * 512, 512)],
                             DL[b], SL[b])
            pltpu.async_copy(hehi_ref.at[pl.ds(ebase0 + b * 512, 512)],
                             DH[b], SL[b])
            pltpu.async_copy(be_ref.at[pl.ds(egrp0 + b * 4, 4)], IX[b], SL[b])

        def epair(p, carry):
            for b in (0, 1):
                j = p * 2 + b
                pltpu.make_async_copy(helo_ref.at[pl.ds(ebase0, 512)],
                                      DL[b], SL[b]).wait()
                pltpu.make_async_copy(hehi_ref.at[pl.ds(ebase0, 512)],
                                      DH[b], SL[b]).wait()
                pltpu.make_async_copy(be_ref.at[pl.ds(egrp0, 4)],
                                      IX[b], SL[b]).wait()
                cps = []
                for kk in range(4):
                    cps.append(pltpu.async_copy(
                        DL[b].at[pl.ds(kk * 128, 128)],
                        sel.at[IX[b].at[kk]], SA[b], add=True))
                    cps.append(pltpu.async_copy(
                        DH[b].at[pl.ds(kk * 128, 128)],
                        seh.at[IX[b].at[kk]], SA[b], add=True))
                    cps.append(pltpu.async_copy(
                        onesv, ce.at[IX[b].at[kk]], SA[b], add=True))
                for cp in cps:
                    cp.wait()
                nj = jnp.minimum(j + 2, _NSC - 1)
                pltpu.async_copy(helo_ref.at[pl.ds(ebase0 + nj * 512, 512)],
                                 DL[b], SL[b])
                pltpu.async_copy(hehi_ref.at[pl.ds(ebase0 + nj * 512, 512)],
                                 DH[b], SL[b])
                pltpu.async_copy(be_ref.at[pl.ds(egrp0 + nj * 4, 4)],
                                 IX[b], SL[b])
            return carry

        lax.fori_loop(0, _NSC // 2, epair, 0)
        for b in (0, 1):
            pltpu.make_async_copy(helo_ref.at[pl.ds(ebase0, 512)],
                                  DL[b], SL[b]).wait()
            pltpu.make_async_copy(hehi_ref.at[pl.ds(ebase0, 512)],
                                  DH[b], SL[b]).wait()
            pltpu.make_async_copy(be_ref.at[pl.ds(egrp0, 4)],
                                  IX[b], SL[b]).wait()

        nbase0 = wid * NPW
        ngrp0 = wid * (NPW // 128)

        def nstep(j, carry):
            pltpu.sync_copy(hn_ref.at[pl.ds(nbase0 + j * 128, 128)], datn)
            pltpu.sync_copy(bn_ref.at[pl.ds(ngrp0 + j, 1)],
                            ix0.at[pl.ds(0, 1)])
            cp1 = pltpu.async_copy(datn, sn.at[ix0.at[0]], sa0, add=True)
            cp2 = pltpu.async_copy(onesv, cn.at[ix0.at[0]], sa0, add=True)
            cp1.wait()
            cp2.wait()
            return carry

        lax.fori_loop(0, NPW // 128, nstep, 0)
        plsc.subcore_barrier()
        rbase = c * G + s * GS
        pltpu.sync_copy(sn.at[pl.ds(s * GS, GS)], o_sn.at[pl.ds(rbase, GS)])
        pltpu.sync_copy(cn.at[pl.ds(s * GS, GS)], o_cn.at[pl.ds(rbase, GS)])
        pltpu.sync_copy(sel.at[pl.ds(s * GS, GS)], o_sel.at[pl.ds(rbase, GS)])
        pltpu.sync_copy(seh.at[pl.ds(s * GS, GS)], o_seh.at[pl.ds(rbase, GS)])
        pltpu.sync_copy(ce.at[pl.ds(s * GS, GS)], o_ce.at[pl.ds(rbase, GS)])

    return k(hn2, he_lo, he_hi, bn2d, be2d, zrows, zrows32, ones)


def kernel(h_node, pos_node, batch_node, h_edge, edge_index, batch_edge,
           Wn, We, Wem, bem, Wnu, bnu, Wf1, bf1, Wf2, bf2):
    src = edge_index[0].astype(I32)
    dst = edge_index[1].astype(I32)
    src_p = jnp.concatenate([src, jnp.zeros((E_PAD - E,), I32)])
    dst_p = jnp.concatenate([dst, jnp.full((E_PAD - E,), N, I32)])
    dloc0 = jnp.where(dst_p < HALF, dst_p, HALF)
    dloc1 = jnp.where((dst_p >= HALF) & (dst_p < N), dst_p - HALF, HALF)
    dloc_flat = jnp.concatenate([dloc0, dloc1]).reshape(2 * (E_PAD // 128),
                                                        128)
    be2d = jnp.concatenate(
        [batch_edge.astype(I32), jnp.full((E_PAD - E,), G, I32)]
    ).reshape(E_PAD // 128, 128)
    bn2d = jnp.concatenate(
        [batch_node.astype(I32), jnp.full((N_PAD - N,), G, I32)]
    ).reshape(N_PAD // 128, 128)

    h_edge_p = jnp.pad(h_edge, ((0, E_PAD - E), (0, 3)))
    h_node_p = jnp.pad(h_node, ((0, N_PAD - N), (0, 0)))
    pos_p = jnp.pad(pos_node, ((0, N_PAD - N), (0, 13)))
    zrows = jnp.zeros((1568, 64), F32)
    zrows32 = jnp.zeros((1568, 32), F32)
    ones = jnp.ones((128, 64), F32)

    We1, Ws1, Wd1 = Wem[0, :64], Wem[0, 64:128], Wem[0, 128:192]
    wd1 = Wem[0, 192:193]
    We2, Ws2, Wd2 = Wem[1, :64], Wem[1, 64:128], Wem[1, 128:192]
    wd2 = Wem[1, 192:193]
    Wh1, Wa1 = Wnu[0, :64], Wnu[0, 64:128]
    Wh2, Wa2 = Wnu[1, :64], Wnu[1, 64:128]
    WeP = jnp.pad(We, ((0, 3), (0, 0)))
    WeF1 = jnp.pad(We @ We1, ((0, 3), (0, 0)))
    be1 = bem[0:1]
    be2 = bem[1:2]
    bn1 = bnu[0:1]
    bn2 = bnu[1:2]
    bf1r = bf1[None, :]
    bf2r = bf2[None, :]

    T1s, T1d = _tc_prep(h_node_p, pos_p, Wn @ Ws1, Wn @ Wd1)
    gA1, gB1 = _sc_gather(
        [(T1s, 80, BF16, 0), (T1d, 80, BF16, 1)], src_p, dst_p)
    he1_lo, he1_hi, dcol = _tc_edge1(h_edge_p, gA1, gB1,
                                     WeP, WeF1, wd1, be1)
    agg1_lo = _sc_scatter(he1_lo, dloc_flat, zrows32)
    agg1_hi = _sc_scatter(he1_hi, dloc_flat, zrows32)
    hn1, T2s, T2d = _tc_node1(h_node_p, agg1_lo, agg1_hi, Wn @ Wh1,
                              Wa1[:32], Wa1[32:], bn1, Wn, Ws2, Wd2)
    gA2, gB2 = _sc_gather([(T2s, 64, BF16, 0), (T2d, 64, BF16, 1)],
                          src_p, dst_p)
    he2_lo, he2_hi = _tc_edge2(he1_lo, he1_hi, gA2, gB2, dcol,
                               We2, wd2, be2)
    agg2_lo = _sc_scatter(he2_lo, dloc_flat, zrows32)
    agg2_hi = _sc_scatter(he2_hi, dloc_flat, zrows32)
    hn2 = _tc_node2(hn1, agg2_lo, agg2_hi, Wh2, Wa2[:32], Wa2[32:], bn2)

    pn, cn, pel, peh, ce = _sc_pool(hn2, he2_lo, he2_hi, bn2d, be2d,
                                    zrows, zrows32, ones)
    emb = _tc_final(pn, cn, pel, peh, ce, Wf1[:64],
                    Wf1[64:96], Wf1[96:], bf1r, Wf2, bf2r)
    return (emb, batch_node)
